# baseline (speedup 1.0000x reference)
import functools

import jax
import jax.numpy as jnp
from jax import lax
from jax.experimental import pallas as pl
from jax.experimental.pallas import tpu as pltpu
from jax.experimental.pallas import tpu_sc as plsc

PAD = 1
LN_EPS = 1e-12
H = 768
L = 16
NH = H // L
NW = 32
R = 32
NBUF = 2


def _pid_body(ids_ref, out_ref):
    ids = ids_ref[...]
    m = (ids != PAD).astype(jnp.int32)
    s = ids.shape[1]
    acc = m
    k = 1
    while k < s:
        z = jnp.zeros((ids.shape[0], k), jnp.int32)
        acc = acc + jnp.concatenate([z, acc[:, : s - k]], axis=1)
        k *= 2
    out_ref[...] = acc * m + PAD


def _position_ids(ids32):
    return pl.pallas_call(
        _pid_body,
        out_shape=jax.ShapeDtypeStruct(ids32.shape, jnp.int32),
    )(ids32)


def _fold_body(pos_ref, type_ref, out_ref):
    out_ref[...] = pos_ref[...] + type_ref[...][0:1, :]


def _fold_type(pos_emb, type_emb):
    n = pos_emb.shape[0]
    blk = 1024
    return pl.pallas_call(
        _fold_body,
        grid=(n // blk,),
        in_specs=[
            pl.BlockSpec((blk, H), lambda i: (i, 0)),
            pl.BlockSpec((2, H), lambda i: (0, 0)),
        ],
        out_specs=pl.BlockSpec((blk, H), lambda i: (i, 0)),
        out_shape=jax.ShapeDtypeStruct((n, H), jnp.float32),
    )(pos_emb, type_emb)


_DNUMS = lax.GatherDimensionNumbers(
    offset_dims=(), collapsed_slice_dims=(0,), start_index_map=(0,))


def _allreduce_sum(x):
    lanes = lax.iota(jnp.int32, L)
    for k in (1, 2, 4, 8):
        idx = lax.bitwise_xor(lanes, jnp.int32(k))
        sh = lax.gather(x, idx[:, None], _DNUMS, slice_sizes=(1,),
                        mode=lax.GatherScatterMode.PROMISE_IN_BOUNDS)
        x = x + sh
    return x


def _rsqrt(x):
    bits = lax.bitcast_convert_type(x, jnp.int32)
    y = lax.bitcast_convert_type(
        jnp.int32(0x5F3759DF) - lax.shift_right_logical(bits, 1), jnp.float32)
    for _ in range(3):
        y = y * (1.5 - 0.5 * x * y * y)
    return y


def _sc_body(word_hbm, pos_hbm, gamma_hbm, beta_hbm, idw_hbm,
             idp_hbm, out_hbm, gamma_v, beta_v, idw_v, idp_v, w_v, p_v, o_v,
             stats_m, stats_y, sem_w, sem_p, sem_o):
    wid = lax.axis_index("s") * 2 + lax.axis_index("c")
    rows_per_w = out_hbm.shape[0] // NW
    nchunk = rows_per_w // R
    base = wid * rows_per_w
    pltpu.sync_copy(gamma_hbm, gamma_v)
    pltpu.sync_copy(beta_hbm, beta_v)

    def start(c, b):
        cb = base + c * R
        pltpu.sync_copy(idw_hbm.at[pl.ds(cb, R)], idw_v.at[b])
        pltpu.sync_copy(idp_hbm.at[pl.ds(cb, R)], idp_v.at[b])
        pltpu.async_copy(word_hbm.at[idw_v.at[b]], w_v.at[b], sem_w)
        pltpu.async_copy(pos_hbm.at[idp_v.at[b]], p_v.at[b], sem_p)

    def wait_bufs(b):
        pltpu.make_async_copy(word_hbm.at[idw_v.at[b]], w_v.at[b],
                              sem_w).wait()
        pltpu.make_async_copy(pos_hbm.at[idp_v.at[b]], p_v.at[b],
                              sem_p).wait()

    def issue_out(c):
        cb = base + c * R
        pltpu.async_copy(o_v, out_hbm.at[pl.ds(cb, R)], sem_o)

    def wait_out():
        pltpu.make_async_copy(o_v, out_hbm.at[pl.ds(0, R)], sem_o).wait()

    def compute(c, b, guard):

        @plsc.parallel_loop(0, R, unroll=2)
        def row(r):
            accs = [jnp.zeros((L,), jnp.float32)] * 8
            for j in range(NH):
                sl = pl.ds(j * L, L)
                t = w_v[b, r, sl] + p_v[b, r, sl]
                accs[j % 4] = accs[j % 4] + t
                accs[4 + j % 4] = accs[4 + j % 4] + t * t
                w_v[b, r, sl] = t
            meanv = (_allreduce_sum(accs[0] + accs[1] + accs[2] + accs[3])
                     * (1.0 / H))
            var = (_allreduce_sum(accs[4] + accs[5] + accs[6] + accs[7])
                   * (1.0 / H) - meanv * meanv + LN_EPS)
            stats_m[r, :] = meanv
            stats_y[r, :] = _rsqrt(var)

        if guard:
            wait_out()

        for g in range(R // L):
            ms = [stats_m[g * L + i, :] for i in range(L)]
            ys = [stats_y[g * L + i, :] for i in range(L)]

            @plsc.parallel_loop(0, NH, unroll=2)
            def col(j):
                sl = pl.ds(pl.multiple_of(j * L, L), L)
                gc = gamma_v[sl]
                bc = beta_v[sl]
                for i in range(L):
                    rr = g * L + i
                    t = w_v[b, rr, sl]
                    o_v[rr, sl] = (t - ms[i]) * ys[i] * gc + bc

    start(0, 0)

    def pair(k, carry):
        c0 = k * 2
        start(c0 + 1, 1)
        wait_bufs(0)
        compute(c0, 0, True)
        issue_out(c0)
        start(jnp.minimum(c0 + 2, nchunk - 2), 0)
        wait_bufs(1)
        compute(c0 + 1, 1, True)
        issue_out(c0 + 1)
        return carry

    wait_bufs(0)
    start(1, 1)
    compute(0, 0, False)
    issue_out(0)
    start(2, 0)
    wait_bufs(1)
    compute(1, 1, True)
    issue_out(1)

    def pair2(k2, carry):
        return pair(k2 + 1, carry)

    lax.fori_loop(0, nchunk // 2 - 1, pair2, 0)
    wait_bufs(0)
    wait_out()


def _sc_call(word_emb, pos2, ln_gamma, ln_beta, idw, idp):
    n = idw.shape[0]
    mesh = plsc.VectorSubcoreMesh(core_axis_name="c", subcore_axis_name="s")
    f = functools.partial(
        pl.kernel,
        mesh=mesh,
        compiler_params=pltpu.CompilerParams(use_tc_tiling_on_sc=False),
        out_type=jax.ShapeDtypeStruct((n, H), jnp.float32),
        scratch_types=[
            pltpu.VMEM((H,), jnp.float32),
            pltpu.VMEM((H,), jnp.float32),
            pltpu.VMEM((NBUF, R), jnp.int32),
            pltpu.VMEM((NBUF, R), jnp.int32),
            pltpu.VMEM((NBUF, R, H), jnp.float32),
            pltpu.VMEM((NBUF, R, H), jnp.float32),
            pltpu.VMEM((R, H), jnp.float32),
            pltpu.VMEM((R, L), jnp.float32),
            pltpu.VMEM((R, L), jnp.float32),
            pltpu.SemaphoreType.DMA,
            pltpu.SemaphoreType.DMA,
            pltpu.SemaphoreType.DMA,
        ],
    )(_sc_body)
    return f(word_emb, pos2, ln_gamma, ln_beta, idw, idp)


def kernel(input_ids, word_emb, type_emb, pos_emb, ln_gamma, ln_beta):
    b, s = input_ids.shape
    ids32 = input_ids.astype(jnp.int32)
    pid = _position_ids(ids32)
    pos2 = _fold_type(pos_emb, type_emb)
    out = _sc_call(word_emb, pos2, ln_gamma, ln_beta,
                   ids32.reshape(-1), pid.reshape(-1))
    return out.reshape(b, s, H)


# speedup vs baseline: 2.6846x; 2.6846x over previous
import functools

import jax
import jax.numpy as jnp
from jax import lax
from jax.experimental import pallas as pl
from jax.experimental.pallas import tpu as pltpu
from jax.experimental.pallas import tpu_sc as plsc

PAD = 1
LN_EPS = 1e-12
H = 768
L = 16
NH = H // L
NW = 32
R = 32
NBUF = 2


def _prep_body(ids_ref, pos_ref, type_ref, pid_ref, pos2_ref):
    pos2_ref[...] = pos_ref[...] + type_ref[...][0:1, :]

    @pl.when(pl.program_id(0) == 0)
    def _():
        ids = ids_ref[...]
        m = (ids != PAD).astype(jnp.int32)
        s = ids.shape[1]
        acc = m
        k = 1
        while k < s:
            z = jnp.zeros((ids.shape[0], k), jnp.int32)
            acc = acc + jnp.concatenate([z, acc[:, : s - k]], axis=1)
            k *= 2
        pid_ref[...] = acc * m + PAD


def _prepass(ids32, pos_emb, type_emb):
    n = pos_emb.shape[0]
    blk = 1024
    return pl.pallas_call(
        _prep_body,
        grid=(n // blk,),
        in_specs=[
            pl.BlockSpec(ids32.shape, lambda i: (0, 0)),
            pl.BlockSpec((blk, H), lambda i: (i, 0)),
            pl.BlockSpec((2, H), lambda i: (0, 0)),
        ],
        out_specs=[
            pl.BlockSpec(ids32.shape, lambda i: (0, 0)),
            pl.BlockSpec((blk, H), lambda i: (i, 0)),
        ],
        out_shape=[
            jax.ShapeDtypeStruct(ids32.shape, jnp.int32),
            jax.ShapeDtypeStruct((n, H), jnp.float32),
        ],
    )(ids32, pos_emb, type_emb)


_DNUMS = lax.GatherDimensionNumbers(
    offset_dims=(), collapsed_slice_dims=(0,), start_index_map=(0,))


def _allreduce_sum(x):
    lanes = lax.iota(jnp.int32, L)
    for k in (1, 2, 4, 8):
        idx = lax.bitwise_xor(lanes, jnp.int32(k))
        sh = lax.gather(x, idx[:, None], _DNUMS, slice_sizes=(1,),
                        mode=lax.GatherScatterMode.PROMISE_IN_BOUNDS)
        x = x + sh
    return x


def _rsqrt(x):
    bits = lax.bitcast_convert_type(x, jnp.int32)
    y = lax.bitcast_convert_type(
        jnp.int32(0x5F3759DF) - lax.shift_right_logical(bits, 1), jnp.float32)
    for _ in range(3):
        y = y * (1.5 - 0.5 * x * y * y)
    return y


def _sc_body(word_hbm, pos_hbm, gamma_hbm, beta_hbm, idw_hbm,
             idp_hbm, out_hbm, gamma_v, beta_v, idw_v, idp_v, w_v, p_v,
             stats_m, stats_y, sem_w, sem_p):
    wid = lax.axis_index("s") * 2 + lax.axis_index("c")
    rows_per_w = out_hbm.shape[0] // NW
    nchunk = rows_per_w // R
    base = wid * rows_per_w
    pltpu.sync_copy(gamma_hbm, gamma_v)
    pltpu.sync_copy(beta_hbm, beta_v)

    def start(c, b):
        cb = base + c * R
        pltpu.sync_copy(idw_hbm.at[pl.ds(cb, R)], idw_v.at[b])
        pltpu.sync_copy(idp_hbm.at[pl.ds(cb, R)], idp_v.at[b])
        pltpu.async_copy(word_hbm.at[idw_v.at[b]], w_v.at[b], sem_w)
        pltpu.async_copy(pos_hbm.at[idp_v.at[b]], p_v.at[b], sem_p)

    def wait_bufs(b):
        pltpu.make_async_copy(word_hbm.at[idw_v.at[b]], w_v.at[b],
                              sem_w).wait()
        pltpu.make_async_copy(pos_hbm.at[idp_v.at[b]], p_v.at[b],
                              sem_p).wait()

    def compute(c, b):
        cb = base + c * R

        @plsc.parallel_loop(0, R, unroll=2)
        def row(r):
            accs = [jnp.zeros((L,), jnp.float32)] * 8
            for j in range(NH):
                sl = pl.ds(j * L, L)
                t = w_v[b, r, sl] + p_v[b, r, sl]
                accs[j % 4] = accs[j % 4] + t
                accs[4 + j % 4] = accs[4 + j % 4] + t * t
                w_v[b, r, sl] = t
            meanv = (_allreduce_sum(accs[0] + accs[1] + accs[2] + accs[3])
                     * (1.0 / H))
            var = (_allreduce_sum(accs[4] + accs[5] + accs[6] + accs[7])
                   * (1.0 / H) - meanv * meanv + LN_EPS)
            stats_m[r, :] = meanv
            stats_y[r, :] = _rsqrt(var)

        for g in range(R // L):
            ms = [stats_m[g * L + i, :] for i in range(L)]
            ys = [stats_y[g * L + i, :] for i in range(L)]

            @plsc.parallel_loop(0, NH, unroll=2)
            def col(j):
                sl = pl.ds(pl.multiple_of(j * L, L), L)
                gc = gamma_v[sl]
                bc = beta_v[sl]
                for i in range(L):
                    rr = g * L + i
                    t = w_v[b, rr, sl]
                    w_v[b, rr, sl] = (t - ms[i]) * ys[i] * gc + bc

        pltpu.sync_copy(w_v.at[b], out_hbm.at[pl.ds(cb, R)])

    start(0, 0)

    def pair(k, carry):
        c0 = k * 2
        start(c0 + 1, 1)
        wait_bufs(0)
        compute(c0, 0)
        start(jnp.minimum(c0 + 2, nchunk - 2), 0)
        wait_bufs(1)
        compute(c0 + 1, 1)
        return carry

    lax.fori_loop(0, nchunk // 2, pair, 0)
    wait_bufs(0)


def _sc_call(word_emb, pos2, ln_gamma, ln_beta, idw, idp):
    n = idw.shape[0]
    mesh = plsc.VectorSubcoreMesh(core_axis_name="c", subcore_axis_name="s")
    f = functools.partial(
        pl.kernel,
        mesh=mesh,
        out_type=jax.ShapeDtypeStruct((n, H), jnp.float32),
        scratch_types=[
            pltpu.VMEM((H,), jnp.float32),
            pltpu.VMEM((H,), jnp.float32),
            pltpu.VMEM((NBUF, R), jnp.int32),
            pltpu.VMEM((NBUF, R), jnp.int32),
            pltpu.VMEM((NBUF, R, H), jnp.float32),
            pltpu.VMEM((NBUF, R, H), jnp.float32),
            pltpu.VMEM((R, L), jnp.float32),
            pltpu.VMEM((R, L), jnp.float32),
            pltpu.SemaphoreType.DMA,
            pltpu.SemaphoreType.DMA,
        ],
    )(_sc_body)
    return f(word_emb, pos2, ln_gamma, ln_beta, idw, idp)


def kernel(input_ids, word_emb, type_emb, pos_emb, ln_gamma, ln_beta):
    b, s = input_ids.shape
    ids32 = input_ids.astype(jnp.int32)
    pid, pos2 = _prepass(ids32, pos_emb, type_emb)
    out = _sc_call(word_emb, pos2, ln_gamma, ln_beta,
                   ids32.reshape(-1), pid.reshape(-1))
    return out.reshape(b, s, H)


# speedup vs baseline: 2.8620x; 1.0661x over previous
import functools

import jax
import jax.numpy as jnp
from jax import lax
from jax.experimental import pallas as pl
from jax.experimental.pallas import tpu as pltpu
from jax.experimental.pallas import tpu_sc as plsc

PAD = 1
LN_EPS = 1e-12
H = 768
L = 16
NH = H // L
NW = 32
R = 32
NBUF = 2


def _prep_body(ids_ref, pos_ref, type_ref, pid_ref, pos2_ref):
    pos2_ref[...] = pos_ref[...] + type_ref[...][0:1, :]

    @pl.when(pl.program_id(0) == 0)
    def _():
        ids = ids_ref[...]
        m = (ids != PAD).astype(jnp.int32)
        s = ids.shape[1]
        acc = m
        k = 1
        while k < s:
            z = jnp.zeros((ids.shape[0], k), jnp.int32)
            acc = acc + jnp.concatenate([z, acc[:, : s - k]], axis=1)
            k *= 2
        pid_ref[...] = acc * m + PAD


def _prepass(ids32, pos_emb, type_emb):
    n = pos_emb.shape[0]
    blk = 1024
    return pl.pallas_call(
        _prep_body,
        grid=(n // blk,),
        in_specs=[
            pl.BlockSpec(ids32.shape, lambda i: (0, 0)),
            pl.BlockSpec((blk, H), lambda i: (i, 0)),
            pl.BlockSpec((2, H), lambda i: (0, 0)),
        ],
        out_specs=[
            pl.BlockSpec(ids32.shape, lambda i: (0, 0)),
            pl.BlockSpec((blk, H), lambda i: (i, 0)),
        ],
        out_shape=[
            jax.ShapeDtypeStruct(ids32.shape, jnp.int32),
            jax.ShapeDtypeStruct((n, H), jnp.float32),
        ],
    )(ids32, pos_emb, type_emb)


_DNUMS = lax.GatherDimensionNumbers(
    offset_dims=(), collapsed_slice_dims=(0,), start_index_map=(0,))


def _allreduce_sum(x):
    lanes = lax.iota(jnp.int32, L)
    for k in (1, 2, 4, 8):
        idx = lax.bitwise_xor(lanes, jnp.int32(k))
        sh = lax.gather(x, idx[:, None], _DNUMS, slice_sizes=(1,),
                        mode=lax.GatherScatterMode.PROMISE_IN_BOUNDS)
        x = x + sh
    return x


def _rsqrt(x):
    bits = lax.bitcast_convert_type(x, jnp.int32)
    y = lax.bitcast_convert_type(
        jnp.int32(0x5F3759DF) - lax.shift_right_logical(bits, 1), jnp.float32)
    for _ in range(2):
        y = y * (1.5 - 0.5 * x * y * y)
    return y


def _sc_body(word_hbm, pos_hbm, gamma_hbm, beta_hbm, idw_hbm,
             idp_hbm, out_hbm, gamma_v, beta_v, idw_v, idp_v, w_v, p_v,
             stats_m, stats_y, sem_w, sem_p):
    wid = lax.axis_index("s") * 2 + lax.axis_index("c")
    rows_per_w = out_hbm.shape[0] // NW
    nchunk = rows_per_w // R
    base = wid * rows_per_w
    pltpu.sync_copy(idw_hbm.at[pl.ds(base, rows_per_w)], idw_v)
    pltpu.sync_copy(idp_hbm.at[pl.ds(base, rows_per_w)], idp_v)
    pltpu.sync_copy(gamma_hbm, gamma_v)
    pltpu.sync_copy(beta_hbm, beta_v)

    def start(c, b):
        pltpu.async_copy(word_hbm.at[idw_v.at[pl.ds(c * R, R)]], w_v.at[b],
                         sem_w)
        pltpu.async_copy(pos_hbm.at[idp_v.at[pl.ds(c * R, R)]], p_v.at[b],
                         sem_p)

    def wait_bufs(b):
        pltpu.make_async_copy(word_hbm.at[idw_v.at[pl.ds(0, R)]], w_v.at[b],
                              sem_w).wait()
        pltpu.make_async_copy(pos_hbm.at[idp_v.at[pl.ds(0, R)]], p_v.at[b],
                              sem_p).wait()

    def compute(c, b):
        cb = base + c * R

        @plsc.parallel_loop(0, R, unroll=2)
        def row(r):
            accs = [jnp.zeros((L,), jnp.float32)] * 8
            for j in range(NH):
                sl = pl.ds(j * L, L)
                t = w_v[b, r, sl] + p_v[b, r, sl]
                accs[j % 4] = accs[j % 4] + t
                accs[4 + j % 4] = accs[4 + j % 4] + t * t
                w_v[b, r, sl] = t
            meanv = (_allreduce_sum(accs[0] + accs[1] + accs[2] + accs[3])
                     * (1.0 / H))
            var = (_allreduce_sum(accs[4] + accs[5] + accs[6] + accs[7])
                   * (1.0 / H) - meanv * meanv + LN_EPS)
            stats_m[r, :] = meanv
            stats_y[r, :] = _rsqrt(var)

        for g in range(R // L):
            ms = [stats_m[g * L + i, :] for i in range(L)]
            ys = [stats_y[g * L + i, :] for i in range(L)]

            @plsc.parallel_loop(0, NH, unroll=2)
            def col(j):
                sl = pl.ds(pl.multiple_of(j * L, L), L)
                gc = gamma_v[sl]
                bc = beta_v[sl]
                for i in range(L):
                    rr = g * L + i
                    t = w_v[b, rr, sl]
                    w_v[b, rr, sl] = (t - ms[i]) * ys[i] * gc + bc

        pltpu.sync_copy(w_v.at[b], out_hbm.at[pl.ds(cb, R)])

    start(0, 0)

    def pair(k, carry):
        c0 = k * 2
        start(c0 + 1, 1)
        wait_bufs(0)
        compute(c0, 0)
        start(jnp.minimum(c0 + 2, nchunk - 2), 0)
        wait_bufs(1)
        compute(c0 + 1, 1)
        return carry

    lax.fori_loop(0, nchunk // 2, pair, 0)
    wait_bufs(0)


def _sc_call(word_emb, pos2, ln_gamma, ln_beta, idw, idp):
    n = idw.shape[0]
    mesh = plsc.VectorSubcoreMesh(core_axis_name="c", subcore_axis_name="s")
    f = functools.partial(
        pl.kernel,
        mesh=mesh,
        out_type=jax.ShapeDtypeStruct((n, H), jnp.float32),
        scratch_types=[
            pltpu.VMEM((H,), jnp.float32),
            pltpu.VMEM((H,), jnp.float32),
            pltpu.VMEM((512, ), jnp.int32),
            pltpu.VMEM((512, ), jnp.int32),
            pltpu.VMEM((NBUF, R, H), jnp.float32),
            pltpu.VMEM((NBUF, R, H), jnp.float32),
            pltpu.VMEM((R, L), jnp.float32),
            pltpu.VMEM((R, L), jnp.float32),
            pltpu.SemaphoreType.DMA,
            pltpu.SemaphoreType.DMA,
        ],
    )(_sc_body)
    return f(word_emb, pos2, ln_gamma, ln_beta, idw, idp)


def kernel(input_ids, word_emb, type_emb, pos_emb, ln_gamma, ln_beta):
    b, s = input_ids.shape
    ids32 = input_ids.astype(jnp.int32)
    pid, pos2 = _prepass(ids32, pos_emb, type_emb)
    out = _sc_call(word_emb, pos2, ln_gamma, ln_beta,
                   ids32.reshape(-1), pid.reshape(-1))
    return out.reshape(b, s, H)


# speedup vs baseline: 2.9257x; 1.0223x over previous
import functools

import jax
import jax.numpy as jnp
from jax import lax
from jax.experimental import pallas as pl
from jax.experimental.pallas import tpu as pltpu
from jax.experimental.pallas import tpu_sc as plsc

PAD = 1
LN_EPS = 1e-12
H = 768
L = 16
NH = H // L
NW = 32
R = 32
NBUF = 2


def _prep_body(ids_ref, pos_ref, type_ref, pid_ref, pos2_ref):
    pos2_ref[...] = pos_ref[...] + type_ref[...][0:1, :]

    @pl.when(pl.program_id(0) == 0)
    def _():
        ids = ids_ref[...]
        m = (ids != PAD).astype(jnp.int32)
        s = ids.shape[1]
        acc = m
        k = 1
        while k < s:
            z = jnp.zeros((ids.shape[0], k), jnp.int32)
            acc = acc + jnp.concatenate([z, acc[:, : s - k]], axis=1)
            k *= 2
        pid_ref[...] = acc * m + PAD


def _prepass(ids32, pos_emb, type_emb):
    n = pos_emb.shape[0]
    blk = 1024
    return pl.pallas_call(
        _prep_body,
        grid=(n // blk,),
        in_specs=[
            pl.BlockSpec(ids32.shape, lambda i: (0, 0)),
            pl.BlockSpec((blk, H), lambda i: (i, 0)),
            pl.BlockSpec((2, H), lambda i: (0, 0)),
        ],
        out_specs=[
            pl.BlockSpec(ids32.shape, lambda i: (0, 0)),
            pl.BlockSpec((blk, H), lambda i: (i, 0)),
        ],
        out_shape=[
            jax.ShapeDtypeStruct(ids32.shape, jnp.int32),
            jax.ShapeDtypeStruct((n, H), jnp.float32),
        ],
    )(ids32, pos_emb, type_emb)


_DNUMS = lax.GatherDimensionNumbers(
    offset_dims=(), collapsed_slice_dims=(0,), start_index_map=(0,))


def _allreduce_sum(x):
    lanes = lax.iota(jnp.int32, L)
    for k in (1, 2, 4, 8):
        idx = lax.bitwise_xor(lanes, jnp.int32(k))
        sh = lax.gather(x, idx[:, None], _DNUMS, slice_sizes=(1,),
                        mode=lax.GatherScatterMode.PROMISE_IN_BOUNDS)
        x = x + sh
    return x


def _rsqrt(x):
    bits = lax.bitcast_convert_type(x, jnp.int32)
    y = lax.bitcast_convert_type(
        jnp.int32(0x5F3759DF) - lax.shift_right_logical(bits, 1), jnp.float32)
    for _ in range(2):
        y = y * (1.5 - 0.5 * x * y * y)
    return y


def _sc_body(word_hbm, pos_hbm, gamma_hbm, beta_hbm, idw_hbm,
             idp_hbm, out_hbm, gamma_v, beta_v, idw_v, idp_v, w_v, p_v,
             stats_m, stats_y, sem_w, sem_p, sem_o):
    wid = lax.axis_index("s") * 2 + lax.axis_index("c")
    rows_per_w = out_hbm.shape[0] // NW
    nchunk = rows_per_w // R
    base = wid * rows_per_w
    pltpu.sync_copy(idw_hbm.at[pl.ds(base, rows_per_w)], idw_v)
    pltpu.sync_copy(idp_hbm.at[pl.ds(base, rows_per_w)], idp_v)
    pltpu.sync_copy(gamma_hbm, gamma_v)
    pltpu.sync_copy(beta_hbm, beta_v)

    def start(c, b):
        pltpu.async_copy(word_hbm.at[idw_v.at[pl.ds(c * R, R)]], w_v.at[b],
                         sem_w)
        pltpu.async_copy(pos_hbm.at[idp_v.at[pl.ds(c * R, R)]], p_v.at[b],
                         sem_p)

    def wait_bufs(b):
        pltpu.make_async_copy(word_hbm.at[idw_v.at[pl.ds(0, R)]], w_v.at[b],
                              sem_w).wait()
        pltpu.make_async_copy(pos_hbm.at[idp_v.at[pl.ds(0, R)]], p_v.at[b],
                              sem_p).wait()

    def compute(c, b):
        cb = base + c * R

        @plsc.parallel_loop(0, R, unroll=2)
        def row(r):
            accs = [jnp.zeros((L,), jnp.float32)] * 8
            for j in range(NH):
                sl = pl.ds(j * L, L)
                t = w_v[b, r, sl] + p_v[b, r, sl]
                accs[j % 4] = accs[j % 4] + t
                accs[4 + j % 4] = accs[4 + j % 4] + t * t
                w_v[b, r, sl] = t
            meanv = (_allreduce_sum(accs[0] + accs[1] + accs[2] + accs[3])
                     * (1.0 / H))
            var = (_allreduce_sum(accs[4] + accs[5] + accs[6] + accs[7])
                   * (1.0 / H) - meanv * meanv + LN_EPS)
            stats_m[r, :] = meanv
            stats_y[r, :] = _rsqrt(var)

        for g in range(R // L):
            ms = [stats_m[g * L + i, :] for i in range(L)]
            ys = [stats_y[g * L + i, :] for i in range(L)]

            @plsc.parallel_loop(0, NH, unroll=2)
            def col(j):
                sl = pl.ds(pl.multiple_of(j * L, L), L)
                gc = gamma_v[sl]
                bc = beta_v[sl]
                for i in range(L):
                    rr = g * L + i
                    t = w_v[b, rr, sl]
                    w_v[b, rr, sl] = (t - ms[i]) * ys[i] * gc + bc

            if g == 0:
                pltpu.async_copy(w_v.at[b].at[pl.ds(0, L)],
                                 out_hbm.at[pl.ds(cb, L)], sem_o)

        pltpu.make_async_copy(w_v.at[b].at[pl.ds(0, L)],
                              out_hbm.at[pl.ds(cb, L)], sem_o).wait()
        pltpu.sync_copy(w_v.at[b].at[pl.ds(L, R - L)],
                        out_hbm.at[pl.ds(cb + L, R - L)])

    start(0, 0)

    def pair(k, carry):
        c0 = k * 2
        start(c0 + 1, 1)
        wait_bufs(0)
        compute(c0, 0)
        start(jnp.minimum(c0 + 2, nchunk - 2), 0)
        wait_bufs(1)
        compute(c0 + 1, 1)
        return carry

    lax.fori_loop(0, nchunk // 2, pair, 0)
    wait_bufs(0)


def _sc_call(word_emb, pos2, ln_gamma, ln_beta, idw, idp):
    n = idw.shape[0]
    mesh = plsc.VectorSubcoreMesh(core_axis_name="c", subcore_axis_name="s")
    f = functools.partial(
        pl.kernel,
        mesh=mesh,
        out_type=jax.ShapeDtypeStruct((n, H), jnp.float32),
        scratch_types=[
            pltpu.VMEM((H,), jnp.float32),
            pltpu.VMEM((H,), jnp.float32),
            pltpu.VMEM((512, ), jnp.int32),
            pltpu.VMEM((512, ), jnp.int32),
            pltpu.VMEM((NBUF, R, H), jnp.float32),
            pltpu.VMEM((NBUF, R, H), jnp.float32),
            pltpu.VMEM((R, L), jnp.float32),
            pltpu.VMEM((R, L), jnp.float32),
            pltpu.SemaphoreType.DMA,
            pltpu.SemaphoreType.DMA,
            pltpu.SemaphoreType.DMA,
        ],
    )(_sc_body)
    return f(word_emb, pos2, ln_gamma, ln_beta, idw, idp)


def kernel(input_ids, word_emb, type_emb, pos_emb, ln_gamma, ln_beta):
    b, s = input_ids.shape
    ids32 = input_ids.astype(jnp.int32)
    pid, pos2 = _prepass(ids32, pos_emb, type_emb)
    out = _sc_call(word_emb, pos2, ln_gamma, ln_beta,
                   ids32.reshape(-1), pid.reshape(-1))
    return out.reshape(b, s, H)
